# baseline (speedup 1.0000x reference)
import jax
import jax.numpy as jnp
from jax.experimental import pallas as pl

_DN_T = (((0,), (0,)), ((), ()))


def _gcn_body(A_ref, x_ref, W1_ref, b1_ref, W2_ref, b2_ref, sp_ref, out_ref):
    A = A_ref[...]
    deg_row = jnp.sum(A, axis=0, keepdims=True) + 1.0
    dis_row = jax.lax.rsqrt(deg_row)
    dis = jnp.transpose(dis_row)

    x = x_ref[...]
    h = jnp.dot(x, W1_ref[...], preferred_element_type=jnp.float32)
    u = dis * h
    t = jax.lax.dot_general(A, u, _DN_T, preferred_element_type=jnp.float32)
    g1 = jnp.maximum(dis * (t + u) + b1_ref[...], 0.0)

    h2 = jnp.dot(g1, W2_ref[...], preferred_element_type=jnp.float32)
    u2 = dis * h2
    t2 = jax.lax.dot_general(A, u2, _DN_T, preferred_element_type=jnp.float32)
    g2 = dis * (t2 + u2) + b2_ref[...] + x

    out_ref[...] = jax.nn.sigmoid(sp_ref[0, 0] * g2)


def kernel(A, x, W1, b1, W2, b2, sigmoid_param):
    n, din = x.shape
    out = pl.pallas_call(
        _gcn_body,
        out_shape=jax.ShapeDtypeStruct((n, din), jnp.float32),
    )(A, x, W1, b1.reshape(1, -1), W2, b2.reshape(1, -1),
      sigmoid_param.reshape(1, 1).astype(jnp.float32))
    return out.astype(jnp.float64)


# speedup vs baseline: 1.1802x; 1.1802x over previous
import jax
import jax.numpy as jnp
from jax.experimental import pallas as pl
from jax.experimental.pallas import tpu as pltpu

_DN_T = (((0,), (0,)), ((), ()))


def _gcn_body(A_ref, x_ref, W1_ref, b1_ref, W2_ref, b2_ref, sp_ref, out_ref,
              A0_s, acc_s):
    i = pl.program_id(0)
    half = A_ref.shape[0]
    blk = A_ref[...]
    colsum = jnp.sum(blk, axis=0, keepdims=True)

    @pl.when(i == 0)
    def _():
        A0_s[...] = blk
        acc_s[...] = colsum

    @pl.when(i == 1)
    def _():
        deg_row = acc_s[...] + colsum + 1.0
        dis_row = jax.lax.rsqrt(deg_row)
        dis = jnp.transpose(dis_row)

        x = x_ref[...]
        h = jnp.dot(x, W1_ref[...], preferred_element_type=jnp.float32)
        u = dis * h
        t = (jax.lax.dot_general(A0_s[...], u[:half], _DN_T,
                                 preferred_element_type=jnp.float32)
             + jax.lax.dot_general(blk, u[half:], _DN_T,
                                   preferred_element_type=jnp.float32))
        g1 = jnp.maximum(dis * (t + u) + b1_ref[...], 0.0)

        h2 = jnp.dot(g1, W2_ref[...], preferred_element_type=jnp.float32)
        u2 = dis * h2
        t2 = (jax.lax.dot_general(A0_s[...], u2[:half], _DN_T,
                                  preferred_element_type=jnp.float32)
              + jax.lax.dot_general(blk, u2[half:], _DN_T,
                                    preferred_element_type=jnp.float32))
        g2 = dis * (t2 + u2) + b2_ref[...] + x

        out_ref[...] = jax.nn.sigmoid(sp_ref[0, 0] * g2)


def kernel(A, x, W1, b1, W2, b2, sigmoid_param):
    n, din = x.shape
    dh = W1.shape[1]
    half = n // 2
    const = lambda i: (0, 0)
    out = pl.pallas_call(
        _gcn_body,
        grid=(2,),
        in_specs=[
            pl.BlockSpec((half, n), lambda i: (i, 0)),
            pl.BlockSpec((n, din), const),
            pl.BlockSpec((din, dh), const),
            pl.BlockSpec((1, dh), const),
            pl.BlockSpec((dh, din), const),
            pl.BlockSpec((1, din), const),
            pl.BlockSpec((1, 1), const),
        ],
        out_specs=pl.BlockSpec((n, din), const),
        out_shape=jax.ShapeDtypeStruct((n, din), jnp.float32),
        scratch_shapes=[
            pltpu.VMEM((half, n), jnp.float32),
            pltpu.VMEM((1, n), jnp.float32),
        ],
    )(A, x, W1, b1.reshape(1, -1), W2, b2.reshape(1, -1),
      sigmoid_param.reshape(1, 1).astype(jnp.float32))
    return out.astype(jnp.float64)
